# baseline (speedup 1.0000x reference)
import functools

import jax
import jax.numpy as jnp
from jax import lax
from jax.experimental import pallas as pl
from jax.experimental.pallas import tpu as pltpu
from jax.experimental.pallas import tpu_sc as plsc

D_MODEL = 128
CHUNK = 128
NBUF = 2


@functools.lru_cache(maxsize=None)
def _build(n_idx: int, n_emb: int):
    info = plsc.get_sparse_core_info()
    nc, ns = info.num_cores, info.num_subcores
    nw = nc * ns
    per_w = n_idx // nw
    n_chunks = per_w // CHUNK
    mesh = plsc.VectorSubcoreMesh(core_axis_name="c", subcore_axis_name="s")

    @functools.partial(
        pl.kernel,
        mesh=mesh,
        out_type=jax.ShapeDtypeStruct((n_idx, D_MODEL), jnp.float32),
        scratch_types=[
            pltpu.VMEM((per_w,), jnp.int32),
            pltpu.VMEM_SHARED((n_emb, D_MODEL), jnp.float32),
            pltpu.VMEM((NBUF, CHUNK, D_MODEL), jnp.float32),
        ]
        + [pltpu.SemaphoreType.DMA] * (2 * NBUF),
    )
    def emb(x_hbm, table_hbm, out_hbm, idx_v, table_sh, rows_v, *sems):
        wid = lax.axis_index("s") * nc + lax.axis_index("c")
        base_w = wid * per_w
        sg = sems[:NBUF]
        sw = sems[NBUF:]

        pltpu.sync_copy(x_hbm.at[pl.ds(base_w, per_w)], idx_v)

        @pl.when(lax.axis_index("s") == 0)
        def _stage_table():
            pltpu.sync_copy(table_hbm, table_sh)

        plsc.subcore_barrier()

        def gather_start(g, b):
            pltpu.async_copy(
                table_sh.at[idx_v.at[pl.ds(g * CHUNK, CHUNK)]], rows_v.at[b], sg[b]
            )

        def gather_wait(b):
            pltpu.make_async_copy(
                table_sh.at[idx_v.at[pl.ds(0, CHUNK)]], rows_v.at[b], sg[b]
            ).wait()

        def write_start(g, b):
            pltpu.async_copy(
                rows_v.at[b], out_hbm.at[pl.ds(base_w + g * CHUNK, CHUNK)], sw[b]
            )

        def write_wait(b):
            pltpu.make_async_copy(
                rows_v.at[b], out_hbm.at[pl.ds(base_w, CHUNK)], sw[b]
            ).wait()

        for b in range(NBUF):
            gather_start(b, b)

        def body(j, carry):
            base_g = NBUF * j
            for b in range(NBUF):
                gather_wait(b)
                write_start(base_g + b, b)
            for b in range(NBUF):
                g = base_g + b

                @pl.when(g + NBUF < n_chunks)
                def _next():
                    write_wait(b)
                    gather_start(g + NBUF, b)

            return carry

        lax.fori_loop(0, n_chunks // NBUF, body, 0)

        for b in range(NBUF):
            write_wait(b)

    return emb


def kernel(x, table):
    b, s = x.shape
    n = b * s
    n_emb = table.shape[0]
    out = _build(n, n_emb)(x.reshape(n).astype(jnp.int32), table.astype(jnp.float32))
    return out.reshape(b, s, D_MODEL)


# speedup vs baseline: 1.3471x; 1.3471x over previous
import functools

import jax
import jax.numpy as jnp
from jax import lax
from jax.experimental import pallas as pl
from jax.experimental.pallas import tpu as pltpu
from jax.experimental.pallas import tpu_sc as plsc

D_MODEL = 128
CHUNK = 128
NBUF = 2


@functools.lru_cache(maxsize=None)
def _build(n_idx: int, n_emb: int):
    info = plsc.get_sparse_core_info()
    nc, ns = info.num_cores, info.num_subcores
    nw = nc * ns
    per_w = n_idx // nw
    n_chunks = per_w // CHUNK
    mesh = plsc.VectorSubcoreMesh(core_axis_name="c", subcore_axis_name="s")

    @functools.partial(
        pl.kernel,
        mesh=mesh,
        out_type=jax.ShapeDtypeStruct((n_idx, D_MODEL), jnp.float32),
        scratch_types=[
            pltpu.VMEM((per_w,), jnp.int32),
            pltpu.VMEM_SHARED((n_emb, D_MODEL), jnp.float32),
            pltpu.VMEM((NBUF, CHUNK, D_MODEL), jnp.float32),
        ]
        + [pltpu.SemaphoreType.DMA] * (2 * NBUF),
    )
    def emb(x_hbm, table_hbm, out_hbm, idx_v, table_sh, rows_v, *sems):
        wid = lax.axis_index("s") * nc + lax.axis_index("c")
        base_w = wid * per_w
        sg = sems[:NBUF]
        sw = sems[NBUF:]

        pltpu.sync_copy(x_hbm.at[pl.ds(base_w, per_w)], idx_v)

        @pl.when(lax.axis_index("s") == 0)
        def _stage_table():
            pltpu.sync_copy(table_hbm, table_sh)

        plsc.subcore_barrier()

        def gather_start(g, b):
            pltpu.async_copy(
                table_sh.at[idx_v.at[pl.ds(g * CHUNK, CHUNK)]], rows_v.at[b], sg[b]
            )

        def gather_wait(b):
            pltpu.make_async_copy(
                table_sh.at[idx_v.at[pl.ds(0, CHUNK)]], rows_v.at[b], sg[b]
            ).wait()

        def write_start(g, b):
            pltpu.async_copy(
                rows_v.at[b], out_hbm.at[pl.ds(base_w + g * CHUNK, CHUNK)], sw[b]
            )

        def write_wait(b):
            pltpu.make_async_copy(
                rows_v.at[b], out_hbm.at[pl.ds(base_w, CHUNK)], sw[b]
            ).wait()

        for b in range(NBUF):
            gather_start(b, b)

        def body(j, carry):
            for b in range(NBUF):
                g = NBUF * j + b
                gather_wait(b)
                write_start(g, b)
                write_wait(b)

                @pl.when(g + NBUF < n_chunks)
                def _next():
                    gather_start(g + NBUF, b)

            return carry

        lax.fori_loop(0, n_chunks // NBUF, body, 0)

    return emb


def kernel(x, table):
    b, s = x.shape
    n = b * s
    n_emb = table.shape[0]
    out = _build(n, n_emb)(x.reshape(n).astype(jnp.int32), table.astype(jnp.float32))
    return out.reshape(b, s, D_MODEL)


# speedup vs baseline: 1.3778x; 1.0228x over previous
import functools

import jax
import jax.numpy as jnp
from jax import lax
from jax.experimental import pallas as pl
from jax.experimental.pallas import tpu as pltpu
from jax.experimental.pallas import tpu_sc as plsc

D_MODEL = 128
CHUNK = 128
NBUF = 4


@functools.lru_cache(maxsize=None)
def _build(n_idx: int, n_emb: int):
    info = plsc.get_sparse_core_info()
    nc, ns = info.num_cores, info.num_subcores
    nw = nc * ns
    per_w = n_idx // nw
    n_chunks = per_w // CHUNK
    mesh = plsc.VectorSubcoreMesh(core_axis_name="c", subcore_axis_name="s")

    @functools.partial(
        pl.kernel,
        mesh=mesh,
        out_type=jax.ShapeDtypeStruct((n_idx, D_MODEL), jnp.float32),
        scratch_types=[
            pltpu.VMEM((per_w,), jnp.int32),
            pltpu.VMEM_SHARED((n_emb, D_MODEL), jnp.float32),
            pltpu.VMEM((NBUF, CHUNK, D_MODEL), jnp.float32),
        ]
        + [pltpu.SemaphoreType.DMA] * (2 * NBUF),
    )
    def emb(x_hbm, table_hbm, out_hbm, idx_v, table_sh, rows_v, *sems):
        wid = lax.axis_index("s") * nc + lax.axis_index("c")
        base_w = wid * per_w
        sg = sems[:NBUF]
        sw = sems[NBUF:]

        pltpu.sync_copy(x_hbm.at[pl.ds(base_w, per_w)], idx_v)

        @pl.when(lax.axis_index("s") == 0)
        def _stage_table():
            pltpu.sync_copy(table_hbm, table_sh)

        plsc.subcore_barrier()

        def gather_start(g, b):
            pltpu.async_copy(
                table_sh.at[idx_v.at[pl.ds(g * CHUNK, CHUNK)]], rows_v.at[b], sg[b]
            )

        def gather_wait(b):
            pltpu.make_async_copy(
                table_sh.at[idx_v.at[pl.ds(0, CHUNK)]], rows_v.at[b], sg[b]
            ).wait()

        def write_start(g, b):
            pltpu.async_copy(
                rows_v.at[b], out_hbm.at[pl.ds(base_w + g * CHUNK, CHUNK)], sw[b]
            )

        def write_wait(b):
            pltpu.make_async_copy(
                rows_v.at[b], out_hbm.at[pl.ds(base_w, CHUNK)], sw[b]
            ).wait()

        for b in range(NBUF):
            gather_start(b, b)

        def body(j, carry):
            for b in range(NBUF):
                g = NBUF * j + b
                gather_wait(b)
                write_start(g, b)
                write_wait(b)

                @pl.when(g + NBUF < n_chunks)
                def _next():
                    gather_start(g + NBUF, b)

            return carry

        lax.fori_loop(0, n_chunks // NBUF, body, 0)

    return emb


def kernel(x, table):
    b, s = x.shape
    n = b * s
    n_emb = table.shape[0]
    out = _build(n, n_emb)(x.reshape(n).astype(jnp.int32), table.astype(jnp.float32))
    return out.reshape(b, s, D_MODEL)


# speedup vs baseline: 1.3869x; 1.0066x over previous
import functools

import jax
import jax.numpy as jnp
from jax import lax
from jax.experimental import pallas as pl
from jax.experimental.pallas import tpu as pltpu
from jax.experimental.pallas import tpu_sc as plsc

D_MODEL = 128
CHUNK = 64
NBUF = 8


@functools.lru_cache(maxsize=None)
def _build(n_idx: int, n_emb: int):
    info = plsc.get_sparse_core_info()
    nc, ns = info.num_cores, info.num_subcores
    nw = nc * ns
    per_w = n_idx // nw
    n_chunks = per_w // CHUNK
    mesh = plsc.VectorSubcoreMesh(core_axis_name="c", subcore_axis_name="s")

    @functools.partial(
        pl.kernel,
        mesh=mesh,
        out_type=jax.ShapeDtypeStruct((n_idx, D_MODEL), jnp.float32),
        scratch_types=[
            pltpu.VMEM((per_w,), jnp.int32),
            pltpu.VMEM_SHARED((n_emb, D_MODEL), jnp.float32),
            pltpu.VMEM((NBUF, CHUNK, D_MODEL), jnp.float32),
        ]
        + [pltpu.SemaphoreType.DMA] * (2 * NBUF),
    )
    def emb(x_hbm, table_hbm, out_hbm, idx_v, table_sh, rows_v, *sems):
        wid = lax.axis_index("s") * nc + lax.axis_index("c")
        base_w = wid * per_w
        sg = sems[:NBUF]
        sw = sems[NBUF:]

        pltpu.sync_copy(x_hbm.at[pl.ds(base_w, per_w)], idx_v)

        @pl.when(lax.axis_index("s") == 0)
        def _stage_table():
            pltpu.sync_copy(table_hbm, table_sh)

        plsc.subcore_barrier()

        def gather_start(g, b):
            pltpu.async_copy(
                table_sh.at[idx_v.at[pl.ds(g * CHUNK, CHUNK)]], rows_v.at[b], sg[b]
            )

        def gather_wait(b):
            pltpu.make_async_copy(
                table_sh.at[idx_v.at[pl.ds(0, CHUNK)]], rows_v.at[b], sg[b]
            ).wait()

        def write_start(g, b):
            pltpu.async_copy(
                rows_v.at[b], out_hbm.at[pl.ds(base_w + g * CHUNK, CHUNK)], sw[b]
            )

        def write_wait(b):
            pltpu.make_async_copy(
                rows_v.at[b], out_hbm.at[pl.ds(base_w, CHUNK)], sw[b]
            ).wait()

        for b in range(NBUF):
            gather_start(b, b)

        def body(j, carry):
            for b in range(NBUF):
                g = NBUF * j + b
                gather_wait(b)
                write_start(g, b)
                write_wait(b)

                @pl.when(g + NBUF < n_chunks)
                def _next():
                    gather_start(g + NBUF, b)

            return carry

        lax.fori_loop(0, n_chunks // NBUF, body, 0)

    return emb


def kernel(x, table):
    b, s = x.shape
    n = b * s
    n_emb = table.shape[0]
    out = _build(n, n_emb)(x.reshape(n).astype(jnp.int32), table.astype(jnp.float32))
    return out.reshape(b, s, D_MODEL)
